# baseline (speedup 1.0000x reference)
import functools

import jax
import jax.numpy as jnp
from jax import lax
from jax.experimental import pallas as pl
from jax.experimental.pallas import tpu as pltpu
from jax.experimental.pallas import tpu_sc as plsc

N = 10000
E = 320000
D_IN = 128
WIDTH = 512
D_OUT = 128

NC = 2
NS = 16
L = 16
NW = NC * NS

K = 128
NB_E = 80
B_TOT = NW * NB_E
E_PAD = B_TOT * K
NB_T = B_TOT // NS
NBUF = 4
N_PAD = 10240
RPT = N_PAD // NS
ZR = 64
DC = 64
DD = 16
DUMMY = N
BN = 256


@functools.cache
def _mesh():
    return plsc.VectorSubcoreMesh(core_axis_name="c", subcore_axis_name="s")


def _fill(ref, rows, cols, val):
    def body(i, carry):
        for j in range(cols // L):
            ref[i, pl.ds(j * L, L)] = jnp.full((L,), val, jnp.float32)
        return carry
    lax.fori_loop(0, rows, body, 0)


def _deg_body(dstb, out, dst_v, ones_v, zeros_v, acc_sh):
    ci = lax.axis_index("c")
    si = lax.axis_index("s")
    w = ci * NS + si
    pltpu.sync_copy(dstb.at[pl.ds(w * NB_E, NB_E)], dst_v)
    _fill(ones_v, K, DD, 1.0)
    _fill(zeros_v, ZR, DD, 0.0)
    for z in range(RPT // ZR):
        pltpu.sync_copy(zeros_v, acc_sh.at[pl.ds(si * RPT + z * ZR, ZR)])
    plsc.subcore_barrier()

    def blk(b, carry):
        pltpu.sync_copy(ones_v, acc_sh.at[dst_v.at[b]], add=True)
        return carry
    lax.fori_loop(0, NB_E, blk, 0)
    plsc.subcore_barrier()
    pltpu.sync_copy(acc_sh.at[pl.ds(si * RPT, RPT)],
                    out.at[ci, pl.ds(si * RPT, RPT)])


@functools.cache
def _sc_degree():
    return pl.kernel(
        _deg_body,
        out_type=jax.ShapeDtypeStruct((NC, N_PAD, DD), jnp.float32),
        mesh=_mesh(),
        compiler_params=pltpu.CompilerParams(use_tc_tiling_on_sc=False),
        scratch_types=[
            pltpu.VMEM((NB_E, K), jnp.int32),
            pltpu.VMEM((K, DD), jnp.float32),
            pltpu.VMEM((ZR, DD), jnp.float32),
            pltpu.VMEM_SHARED((N_PAD, DD), jnp.float32),
        ],
    )


def _chunk_split(n_chunks):
    n0 = (n_chunks + 1) // 2
    return tuple(range(n0)), tuple(range(n0, n_chunks))


@functools.cache
def _make_sc_scatter(n_chunks):
    def body(*args):
        tables = args[:n_chunks]
        srcb = args[n_chunks]
        dstb = args[n_chunks + 1]
        outs = args[n_chunks + 2:2 * n_chunks + 2]
        rest = args[2 * n_chunks + 2:]
        src_v, dst_v = rest[0], rest[1]
        rows = rest[2:2 + NBUF]
        zeros_v, acc_sh = rest[2 + NBUF], rest[3 + NBUF]
        sems = rest[4 + NBUF:4 + 2 * NBUF]
        ssems = rest[4 + 2 * NBUF:4 + 3 * NBUF]
        ci = lax.axis_index("c")
        si = lax.axis_index("s")
        pltpu.sync_copy(srcb.at[pl.ds(si * NB_T, NB_T)], src_v)
        pltpu.sync_copy(dstb.at[pl.ds(si * NB_T, NB_T)], dst_v)
        _fill(zeros_v, ZR, DC, 0.0)

        def chunk_pass(table):
            def g_wait(b, i):
                pltpu.make_async_copy(table.at[src_v.at[b]],
                                      rows[i], sems[i]).wait()

            def s_wait(b, i):
                pltpu.make_async_copy(rows[i],
                                      acc_sh.at[dst_v.at[b]], ssems[i]).wait()

            pltpu.async_copy(table.at[src_v.at[0]], rows[0], sems[0])
            pltpu.async_copy(table.at[src_v.at[1]], rows[1], sems[1])
            for b in (0, 1):
                g_wait(b, b)
                pltpu.async_copy(rows[b], acc_sh.at[dst_v.at[b]], ssems[b],
                                 add=True)
                pltpu.async_copy(table.at[src_v.at[b + 2]],
                                 rows[b + 2], sems[b + 2])

            def lp(t, carry):
                j = 4 * t + 2
                for q in range(4):
                    b = j + q
                    i = (2 + q) % 4
                    i2 = q
                    g_wait(b, i)
                    pltpu.async_copy(rows[i], acc_sh.at[dst_v.at[b]],
                                     ssems[i], add=True)
                    s_wait(b - 2, i2)
                    pltpu.async_copy(table.at[src_v.at[b + 2]],
                                     rows[i2], sems[i2])
                return carry
            lax.fori_loop(0, (NB_T - 4) // 4, lp, 0)

            for b in (NB_T - 2, NB_T - 1):
                i = b % 4
                g_wait(b, i)
                pltpu.async_copy(rows[i], acc_sh.at[dst_v.at[b]], ssems[i],
                                 add=True)
                s_wait(b - 2, (b + 2) % 4)
            for b in (NB_T - 2, NB_T - 1):
                s_wait(b, b % 4)

        def core_pass(chunks):
            for c in chunks:
                for z in range(RPT // ZR):
                    pltpu.sync_copy(zeros_v,
                                    acc_sh.at[pl.ds(si * RPT + z * ZR, ZR)])
                plsc.subcore_barrier()
                chunk_pass(tables[c])
                plsc.subcore_barrier()
                pltpu.sync_copy(acc_sh.at[pl.ds(si * RPT, RPT)],
                                outs[c].at[pl.ds(si * RPT, RPT)])

        ch0, ch1 = _chunk_split(n_chunks)

        @pl.when(ci == 0)
        def _():
            core_pass(ch0)

        @pl.when(ci == 1)
        def _():
            core_pass(ch1)

    return pl.kernel(
        body,
        out_type=[jax.ShapeDtypeStruct((N_PAD, DC), jnp.float32)] * n_chunks,
        mesh=_mesh(),
        compiler_params=pltpu.CompilerParams(use_tc_tiling_on_sc=False),
        scratch_types=[
            pltpu.VMEM((NB_T, K), jnp.int32),
            pltpu.VMEM((NB_T, K), jnp.int32),
            *([pltpu.VMEM((K, DC), jnp.float32)] * NBUF),
            pltpu.VMEM((ZR, DC), jnp.float32),
            pltpu.VMEM_SHARED((N_PAD, DC), jnp.float32),
            *([pltpu.SemaphoreType.DMA] * (2 * NBUF)),
        ],
    )


def _dinv_of(degp_ref):
    deg = degp_ref[0, :, 0:1] + degp_ref[1, :, 0:1]
    return lax.rsqrt(deg + 1.0)


def _mm1_body(x_ref, w_ref, degp_ref, *out_refs):
    dinv = _dinv_of(degp_ref)
    h = jnp.dot(x_ref[...], w_ref[...], preferred_element_type=jnp.float32)
    g = h * dinv
    for c in range(len(out_refs)):
        out_refs[c][...] = g[:, c * DC:(c + 1) * DC]


def _ep_mm_body(n_in, n_out, degp_ref, b_ref, w_ref, *rest):
    acc_refs = rest[:n_in]
    g_refs = rest[n_in:2 * n_in]
    out_refs = rest[2 * n_in:]
    dinv = _dinv_of(degp_ref)
    u = None
    for p in range(n_in // 2):
        hs = []
        for q in range(2):
            c = 2 * p + q
            a = acc_refs[c][...] + g_refs[c][...]
            hs.append(jnp.tanh(a * dinv + b_ref[p, pl.ds(q * DC, DC)][None, :]))
        hc = jnp.concatenate(hs, axis=1)
        part = jnp.dot(hc, w_ref[p, :, :], preferred_element_type=jnp.float32)
        u = part if u is None else u + part
    g = u * dinv
    for c in range(n_out):
        out_refs[c][...] = g[:, c * DC:(c + 1) * DC]


def _ep_final_body(degp_ref, b_ref, a0_ref, a1_ref, g0_ref, g1_ref, out_ref):
    dinv = _dinv_of(degp_ref)
    for c, (a_ref, g_ref) in enumerate(((a0_ref, g0_ref), (a1_ref, g1_ref))):
        s = a_ref[...] + g_ref[...]
        out_ref[:, pl.ds(c * DC, DC)] = \
            s * dinv + b_ref[0, pl.ds(c * DC, DC)][None, :]


_GRID = (N_PAD // BN,)
_spec_out = pl.BlockSpec((BN, DC), lambda i: (i, 0))
_spec_degp = pl.BlockSpec((NC, BN, DD), lambda i: (0, i, 0))
_spec_acc = pl.BlockSpec((BN, DC), lambda i: (i, 0))


def _mm1(x_pad, W1, degp):
    return pl.pallas_call(
        _mm1_body,
        grid=_GRID,
        in_specs=[
            pl.BlockSpec((BN, D_IN), lambda i: (i, 0)),
            pl.BlockSpec((D_IN, WIDTH), lambda i: (0, 0)),
            _spec_degp,
        ],
        out_specs=[_spec_out] * (WIDTH // DC),
        out_shape=[jax.ShapeDtypeStruct((N_PAD, DC), jnp.float32)] * (WIDTH // DC),
    )(x_pad, W1, degp)


def _ep_mm(n_in, n_out, degp, b, w, accs, gs):
    return pl.pallas_call(
        functools.partial(_ep_mm_body, n_in, n_out),
        grid=_GRID,
        in_specs=[
            _spec_degp,
            pl.BlockSpec(b.shape, lambda i: (0, 0)),
            pl.BlockSpec(w.shape, lambda i: (0, 0, 0)),
            *([_spec_acc] * n_in),
            *([_spec_out] * n_in),
        ],
        out_specs=[_spec_out] * n_out,
        out_shape=[jax.ShapeDtypeStruct((N_PAD, DC), jnp.float32)] * n_out,
    )(degp, b, w, *accs, *gs)


def _ep_final(degp, b3, a3, g3):
    return pl.pallas_call(
        _ep_final_body,
        grid=_GRID,
        in_specs=[
            _spec_degp,
            pl.BlockSpec((1, D_OUT), lambda i: (0, 0)),
            _spec_acc,
            _spec_acc,
            _spec_out,
            _spec_out,
        ],
        out_specs=pl.BlockSpec((BN, D_OUT), lambda i: (i, 0)),
        out_shape=jax.ShapeDtypeStruct((N_PAD, D_OUT), jnp.float32),
    )(degp, b3, a3[0], a3[1], g3[0], g3[1])


def kernel(x, edge_index, W1, b1, W2, b2, W3, b3):
    src = edge_index[0].astype(jnp.int32)
    dst = edge_index[1].astype(jnp.int32)
    pad = jnp.full((E_PAD - E,), DUMMY, jnp.int32)
    src_b = jnp.concatenate([src, pad]).reshape(B_TOT, K)
    dst_b = jnp.concatenate([dst, pad]).reshape(B_TOT, K)
    x_pad = jnp.zeros((N_PAD, D_IN), jnp.float32).at[:N].set(x)

    degp = _sc_degree()(dst_b)
    g1 = _mm1(x_pad, W1, degp)
    a1 = _make_sc_scatter(8)(*g1, src_b, dst_b)
    g2 = _ep_mm(8, 8, degp, b1.reshape(4, 128), W2.reshape(4, 128, WIDTH),
                a1, g1)
    a2 = _make_sc_scatter(8)(*g2, src_b, dst_b)
    g3 = _ep_mm(8, 2, degp, b2.reshape(4, 128), W3.reshape(4, 128, D_OUT),
                a2, g2)
    a3 = _make_sc_scatter(2)(*g3, src_b, dst_b)
    out = _ep_final(degp, b3.reshape(1, 128), a3, g3)
    return out[:N]


# speedup vs baseline: 1.0417x; 1.0417x over previous
import functools

import jax
import jax.numpy as jnp
from jax import lax
from jax.experimental import pallas as pl
from jax.experimental.pallas import tpu as pltpu
from jax.experimental.pallas import tpu_sc as plsc

N = 10000
E = 320000
D_IN = 128
WIDTH = 512
D_OUT = 128

NC = 2
NS = 16
L = 16
NW = NC * NS

K = 128
NB_E = 80
B_TOT = NW * NB_E
E_PAD = B_TOT * K
NB_T = B_TOT // NS
NBUF = 4
N_PAD = 10240
RPT = N_PAD // NS
ZR = 64
DC = 64
DD = 16
DUMMY = N
BN = 256


@functools.cache
def _mesh():
    return plsc.VectorSubcoreMesh(core_axis_name="c", subcore_axis_name="s")


def _fill(ref, rows, cols, val):
    def body(i, carry):
        for j in range(cols // L):
            ref[i, pl.ds(j * L, L)] = jnp.full((L,), val, jnp.float32)
        return carry
    lax.fori_loop(0, rows, body, 0)


def _deg_body(dstb, out, dst_v, ones_v, zeros_v, acc_sh):
    ci = lax.axis_index("c")
    si = lax.axis_index("s")
    w = ci * NS + si
    pltpu.sync_copy(dstb.at[pl.ds(w * NB_E, NB_E)], dst_v)
    _fill(ones_v, K, DD, 1.0)
    _fill(zeros_v, ZR, DD, 0.0)
    for z in range(RPT // ZR):
        pltpu.sync_copy(zeros_v, acc_sh.at[pl.ds(si * RPT + z * ZR, ZR)])
    plsc.subcore_barrier()

    def blk(b, carry):
        pltpu.sync_copy(ones_v, acc_sh.at[dst_v.at[b]], add=True)
        return carry
    lax.fori_loop(0, NB_E, blk, 0)
    plsc.subcore_barrier()
    pltpu.sync_copy(acc_sh.at[pl.ds(si * RPT, RPT)],
                    out.at[ci, pl.ds(si * RPT, RPT)])


@functools.cache
def _sc_degree():
    return pl.kernel(
        _deg_body,
        out_type=jax.ShapeDtypeStruct((NC, N_PAD, DD), jnp.float32),
        mesh=_mesh(),
        compiler_params=pltpu.CompilerParams(use_tc_tiling_on_sc=False),
        scratch_types=[
            pltpu.VMEM((NB_E, K), jnp.int32),
            pltpu.VMEM((K, DD), jnp.float32),
            pltpu.VMEM((ZR, DD), jnp.float32),
            pltpu.VMEM_SHARED((N_PAD, DD), jnp.float32),
        ],
    )


def _chunk_split(n_chunks):
    n0 = (n_chunks + 1) // 2
    return tuple(range(n0)), tuple(range(n0, n_chunks))


@functools.cache
def _make_sc_scatter(n_chunks):
    def body(*args):
        tables = args[:n_chunks]
        srcb = args[n_chunks]
        dstb = args[n_chunks + 1]
        outs = args[n_chunks + 2:2 * n_chunks + 2]
        rest = args[2 * n_chunks + 2:]
        src_v, dst_v = rest[0], rest[1]
        rows = rest[2:2 + NBUF]
        zeros_v, acc_sh = rest[2 + NBUF], rest[3 + NBUF]
        sems = rest[4 + NBUF:4 + 2 * NBUF]
        ci = lax.axis_index("c")
        si = lax.axis_index("s")
        pltpu.sync_copy(srcb.at[pl.ds(si * NB_T, NB_T)], src_v)
        pltpu.sync_copy(dstb.at[pl.ds(si * NB_T, NB_T)], dst_v)
        _fill(zeros_v, ZR, DC, 0.0)

        def chunk_pass(table):
            for b in range(NBUF):
                pltpu.async_copy(table.at[src_v.at[b]], rows[b], sems[b])

            def lp(t, carry):
                j = NBUF * t
                for b in range(NBUF):
                    pltpu.make_async_copy(table.at[src_v.at[j + b]],
                                          rows[b], sems[b]).wait()
                    pltpu.sync_copy(rows[b], acc_sh.at[dst_v.at[j + b]],
                                    add=True)
                    pltpu.async_copy(table.at[src_v.at[j + b + NBUF]],
                                     rows[b], sems[b])
                return carry
            lax.fori_loop(0, (NB_T - NBUF) // NBUF, lp, 0)

            for b in range(NBUF):
                jb = NB_T - NBUF + b
                pltpu.make_async_copy(table.at[src_v.at[jb]],
                                      rows[b], sems[b]).wait()
                pltpu.sync_copy(rows[b], acc_sh.at[dst_v.at[jb]], add=True)

        def core_pass(chunks):
            for c in chunks:
                for z in range(RPT // ZR):
                    pltpu.sync_copy(zeros_v,
                                    acc_sh.at[pl.ds(si * RPT + z * ZR, ZR)])
                plsc.subcore_barrier()
                chunk_pass(tables[c])
                plsc.subcore_barrier()
                pltpu.sync_copy(acc_sh.at[pl.ds(si * RPT, RPT)],
                                outs[c].at[pl.ds(si * RPT, RPT)])

        ch0, ch1 = _chunk_split(n_chunks)

        @pl.when(ci == 0)
        def _():
            core_pass(ch0)

        @pl.when(ci == 1)
        def _():
            core_pass(ch1)

    return pl.kernel(
        body,
        out_type=[jax.ShapeDtypeStruct((N_PAD, DC), jnp.float32)] * n_chunks,
        mesh=_mesh(),
        compiler_params=pltpu.CompilerParams(use_tc_tiling_on_sc=False),
        scratch_types=[
            pltpu.VMEM((NB_T, K), jnp.int32),
            pltpu.VMEM((NB_T, K), jnp.int32),
            *([pltpu.VMEM((K, DC), jnp.float32)] * NBUF),
            pltpu.VMEM((ZR, DC), jnp.float32),
            pltpu.VMEM_SHARED((N_PAD, DC), jnp.float32),
            *([pltpu.SemaphoreType.DMA] * NBUF),
        ],
    )


def _dinv_of(degp_ref):
    deg = degp_ref[0, :, 0:1] + degp_ref[1, :, 0:1]
    return lax.rsqrt(deg + 1.0)


def _mm1_body(x_ref, w_ref, degp_ref, *out_refs):
    dinv = _dinv_of(degp_ref)
    h = jnp.dot(x_ref[...], w_ref[...], preferred_element_type=jnp.float32)
    g = h * dinv
    for c in range(len(out_refs)):
        out_refs[c][...] = g[:, c * DC:(c + 1) * DC]


def _ep_mm_body(n_in, n_out, degp_ref, b_ref, w_ref, *rest):
    acc_refs = rest[:n_in]
    g_refs = rest[n_in:2 * n_in]
    out_refs = rest[2 * n_in:]
    dinv = _dinv_of(degp_ref)
    u = None
    for p in range(n_in // 2):
        hs = []
        for q in range(2):
            c = 2 * p + q
            a = acc_refs[c][...] + g_refs[c][...]
            hs.append(jnp.tanh(a * dinv + b_ref[p, pl.ds(q * DC, DC)][None, :]))
        hc = jnp.concatenate(hs, axis=1)
        part = jnp.dot(hc, w_ref[p, :, :], preferred_element_type=jnp.float32)
        u = part if u is None else u + part
    g = u * dinv
    for c in range(n_out):
        out_refs[c][...] = g[:, c * DC:(c + 1) * DC]


def _ep_final_body(degp_ref, b_ref, a0_ref, a1_ref, g0_ref, g1_ref, out_ref):
    dinv = _dinv_of(degp_ref)
    for c, (a_ref, g_ref) in enumerate(((a0_ref, g0_ref), (a1_ref, g1_ref))):
        s = a_ref[...] + g_ref[...]
        out_ref[:, pl.ds(c * DC, DC)] = \
            s * dinv + b_ref[0, pl.ds(c * DC, DC)][None, :]


_GRID = (N_PAD // BN,)
_spec_out = pl.BlockSpec((BN, DC), lambda i: (i, 0))
_spec_degp = pl.BlockSpec((NC, BN, DD), lambda i: (0, i, 0))
_spec_acc = pl.BlockSpec((BN, DC), lambda i: (i, 0))


def _mm1(x_pad, W1, degp):
    return pl.pallas_call(
        _mm1_body,
        grid=_GRID,
        in_specs=[
            pl.BlockSpec((BN, D_IN), lambda i: (i, 0)),
            pl.BlockSpec((D_IN, WIDTH), lambda i: (0, 0)),
            _spec_degp,
        ],
        out_specs=[_spec_out] * (WIDTH // DC),
        out_shape=[jax.ShapeDtypeStruct((N_PAD, DC), jnp.float32)] * (WIDTH // DC),
    )(x_pad, W1, degp)


def _ep_mm(n_in, n_out, degp, b, w, accs, gs):
    return pl.pallas_call(
        functools.partial(_ep_mm_body, n_in, n_out),
        grid=_GRID,
        in_specs=[
            _spec_degp,
            pl.BlockSpec(b.shape, lambda i: (0, 0)),
            pl.BlockSpec(w.shape, lambda i: (0, 0, 0)),
            *([_spec_acc] * n_in),
            *([_spec_out] * n_in),
        ],
        out_specs=[_spec_out] * n_out,
        out_shape=[jax.ShapeDtypeStruct((N_PAD, DC), jnp.float32)] * n_out,
    )(degp, b, w, *accs, *gs)


def _ep_final(degp, b3, a3, g3):
    return pl.pallas_call(
        _ep_final_body,
        grid=_GRID,
        in_specs=[
            _spec_degp,
            pl.BlockSpec((1, D_OUT), lambda i: (0, 0)),
            _spec_acc,
            _spec_acc,
            _spec_out,
            _spec_out,
        ],
        out_specs=pl.BlockSpec((BN, D_OUT), lambda i: (i, 0)),
        out_shape=jax.ShapeDtypeStruct((N_PAD, D_OUT), jnp.float32),
    )(degp, b3, a3[0], a3[1], g3[0], g3[1])


def kernel(x, edge_index, W1, b1, W2, b2, W3, b3):
    src = edge_index[0].astype(jnp.int32)
    dst = edge_index[1].astype(jnp.int32)
    pad = jnp.full((E_PAD - E,), DUMMY, jnp.int32)
    src_b = jnp.concatenate([src, pad]).reshape(B_TOT, K)
    dst_b = jnp.concatenate([dst, pad]).reshape(B_TOT, K)
    x_pad = jnp.zeros((N_PAD, D_IN), jnp.float32).at[:N].set(x)

    degp = _sc_degree()(dst_b)
    g1 = _mm1(x_pad, W1, degp)
    a1 = _make_sc_scatter(8)(*g1, src_b, dst_b)
    g2 = _ep_mm(8, 8, degp, b1.reshape(4, 128), W2.reshape(4, 128, WIDTH),
                a1, g1)
    a2 = _make_sc_scatter(8)(*g2, src_b, dst_b)
    g3 = _ep_mm(8, 2, degp, b2.reshape(4, 128), W3.reshape(4, 128, D_OUT),
                a2, g2)
    a3 = _make_sc_scatter(2)(*g3, src_b, dst_b)
    out = _ep_final(degp, b3.reshape(1, 128), a3, g3)
    return out[:N]
